# baseline (speedup 1.0000x reference)
import functools

import jax
import jax.numpy as jnp
from jax import lax
from jax.experimental import pallas as pl
from jax.experimental.pallas import tpu as pltpu
from jax.experimental.pallas import tpu_sc as plsc

NUM_FIELDS = 26
VOCAB = 100000
EMB = 5
B = 4096
NUM_DENSE = 13
OUTW = NUM_FIELDS * EMB + NUM_DENSE

VPAD = 100352
FSTRIDE = VPAD
ESTRIDE = NUM_FIELDS * VPAD
NWORDS = EMB * ESTRIDE

NUM_CORES = 2
NUM_WORKERS = 32
BPW = B // NUM_WORKERS
JPW = BPW * NUM_FIELDS
CHUNK = 128
GPW = JPW * EMB
OPW = BPW * OUTW
NGATHER = GPW // CHUNK
NIDXV = JPW // 16
NGV = GPW // 16
NBCHUNK = BPW // 16


def _sc_body(ftab_hbm, idx_hbm, numt_hbm, w_hbm, b_hbm, out_hbm,
             idxv, eidx, vals, nf, wv, bv, outb, sem):
    wid = lax.axis_index("s") * NUM_CORES + lax.axis_index("c")
    base = wid * JPW
    b0 = wid * BPW

    pltpu.sync_copy(idx_hbm.at[pl.ds(base, JPW)], idxv)
    pltpu.sync_copy(numt_hbm.at[:, pl.ds(b0, BPW)], nf)
    pltpu.sync_copy(w_hbm, wv)
    pltpu.sync_copy(b_hbm, bv)

    lanes = lax.iota(jnp.int32, 16)

    def add_offsets(i, _):
        off16 = i * 16
        j = lanes + off16
        f = lax.rem(j, NUM_FIELDS)
        idxv[pl.ds(off16, 16)] = idxv[pl.ds(off16, 16)] + f * FSTRIDE
        return 0

    lax.fori_loop(0, NIDXV, add_offsets, 0)

    def expand(i, _):
        p0 = i * 16
        p = lanes + p0
        j = lax.div(p, EMB)
        e = p - j * EMB
        eidx[pl.ds(p0, 16)] = plsc.load_gather(idxv, [j]) + e * ESTRIDE
        return 0

    lax.fori_loop(0, NGV, expand, 0)

    copies = []
    for k in range(NGATHER):
        copies.append(
            pltpu.async_copy(
                ftab_hbm.at[eidx.at[pl.ds(k * CHUNK, CHUNK)]],
                vals.at[pl.ds(k * CHUNK, CHUNK)],
                sem,
            )
        )

    def dense_chunk(c, _):
        c16 = c * 16
        dst = (lanes + c16) * OUTW + NUM_FIELDS * EMB
        for j in range(NUM_DENSE):
            acc = bv[pl.ds(j * 16, 16)]
            for k in range(NUM_DENSE):
                acc = acc + nf[k, pl.ds(c16, 16)] * wv[pl.ds((j * NUM_DENSE + k) * 16, 16)]
            plsc.store_scatter(outb, [dst + j], acc)
        return 0

    lax.fori_loop(0, NBCHUNK, dense_chunk, 0)

    for c in copies:
        c.wait()

    def place(i, _):
        p0 = i * 16
        p = lanes + p0
        pos = p + NUM_DENSE * lax.div(p, NUM_FIELDS * EMB)
        plsc.store_scatter(outb, [pos], vals[pl.ds(p0, 16)])
        return 0

    lax.fori_loop(0, NGV, place, 0)

    pltpu.sync_copy(outb, out_hbm.at[pl.ds(wid * OPW, OPW)])


def _sc_kernel(ftab, idx_flat, num_t, w_splat, b_splat):
    mesh = plsc.VectorSubcoreMesh(core_axis_name="c", subcore_axis_name="s")
    kern = functools.partial(
        pl.kernel,
        mesh=mesh,
        out_type=jax.ShapeDtypeStruct((B * OUTW,), jnp.float32),
        scratch_types=[
            pltpu.VMEM((JPW,), jnp.int32),
            pltpu.VMEM((GPW,), jnp.int32),
            pltpu.VMEM((GPW,), jnp.float32),
            pltpu.VMEM((NUM_DENSE, BPW), jnp.float32),
            pltpu.VMEM((NUM_DENSE * NUM_DENSE * 16,), jnp.float32),
            pltpu.VMEM((NUM_DENSE * 16,), jnp.float32),
            pltpu.VMEM((OPW,), jnp.float32),
            pltpu.SemaphoreType.DMA,
        ],
        compiler_params=pltpu.CompilerParams(
            use_tc_tiling_on_sc=False, needs_layout_passes=False),
    )(_sc_body)
    return kern(ftab, idx_flat, num_t, w_splat, b_splat)


def _untile_body(x_ref, o_ref):
    for f in range(NUM_FIELDS):
        o_ref[pl.ds(f * VPAD, VOCAB)] = x_ref[0, f, :]


def _untile(tables_ev):
    return pl.pallas_call(
        _untile_body,
        grid=(EMB,),
        in_specs=[pl.BlockSpec((1, NUM_FIELDS, VOCAB), lambda e: (e, 0, 0))],
        out_specs=pl.BlockSpec((ESTRIDE,), lambda e: (e,)),
        out_shape=jax.ShapeDtypeStruct((NWORDS,), jnp.float32),
    )(tables_ev)


def kernel(cate_feat, num_feat, tables, W, b):
    ftab = _untile(jnp.transpose(tables, (2, 0, 1)))
    idx_flat = cate_feat.astype(jnp.int32).T.reshape(-1)
    num_t = num_feat.T
    w_splat = jnp.repeat(W.reshape(-1, 1), 16, axis=1).reshape(-1)
    b_splat = jnp.repeat(b.reshape(-1, 1), 16, axis=1).reshape(-1)
    out = _sc_kernel(ftab, idx_flat, num_t, w_splat, b_splat)
    return out.reshape(B, OUTW)


# speedup vs baseline: 1.0720x; 1.0720x over previous
import functools

import jax
import jax.numpy as jnp
from jax import lax
from jax.experimental import pallas as pl
from jax.experimental.pallas import tpu as pltpu
from jax.experimental.pallas import tpu_sc as plsc

NUM_FIELDS = 26
VOCAB = 100000
EMB = 5
B = 4096
NUM_DENSE = 13

NUM_CORES = 2
NUM_WORKERS = 32
BPW = B // NUM_WORKERS
JPW = BPW * NUM_FIELDS
VPAD = 100352
FSTRIDE = VPAD
ESTRIDE = NUM_FIELDS * VPAD
NWORDS = EMB * ESTRIDE
CHUNK = 128
OPW = JPW * EMB
NGATHER = OPW // CHUNK
NIDXV = JPW // 16
NOUTV = OPW // 16


def _sc_gather_body(ftab_hbm, idx_hbm, out_hbm, idxv, eidx, vals, sem):
    wid = lax.axis_index("s") * NUM_CORES + lax.axis_index("c")
    base = wid * JPW

    pltpu.sync_copy(idx_hbm.at[pl.ds(base, JPW)], idxv)

    lanes = lax.iota(jnp.int32, 16)

    def add_offsets(i, _):
        off16 = i * 16
        j = lanes + off16
        f = lax.rem(j, NUM_FIELDS)
        idxv[pl.ds(off16, 16)] = idxv[pl.ds(off16, 16)] + f * FSTRIDE
        return 0

    lax.fori_loop(0, NIDXV, add_offsets, 0)

    def expand(i, _):
        p0 = i * 16
        p = lanes + p0
        j = lax.div(p, EMB)
        e = p - j * EMB
        eidx[pl.ds(p0, 16)] = plsc.load_gather(idxv, [j]) + e * ESTRIDE
        return 0

    copies = []
    half = NOUTV // 2
    for g in range(2):
        lax.fori_loop(g * half, (g + 1) * half, expand, 0)
        for k in range(g * NGATHER // 2, (g + 1) * NGATHER // 2):
            copies.append(
                pltpu.async_copy(
                    ftab_hbm.at[eidx.at[pl.ds(k * CHUNK, CHUNK)]],
                    vals.at[pl.ds(k * CHUNK, CHUNK)],
                    sem,
                )
            )
    for c in copies:
        c.wait()

    pltpu.sync_copy(vals, out_hbm.at[pl.ds(wid * OPW, OPW)])


def _sc_gather(ftab, idx_flat):
    mesh = plsc.VectorSubcoreMesh(core_axis_name="c", subcore_axis_name="s")
    kern = functools.partial(
        pl.kernel,
        mesh=mesh,
        out_type=jax.ShapeDtypeStruct((B * NUM_FIELDS * EMB,), jnp.float32),
        scratch_types=[
            pltpu.VMEM((JPW,), jnp.int32),
            pltpu.VMEM((OPW,), jnp.int32),
            pltpu.VMEM((OPW,), jnp.float32),
            pltpu.SemaphoreType.DMA,
        ],
        compiler_params=pltpu.CompilerParams(
            use_tc_tiling_on_sc=False, needs_layout_passes=False),
    )(_sc_gather_body)
    return kern(ftab, idx_flat)


def _untile_body(x_ref, o_ref):
    for f in range(NUM_FIELDS):
        o_ref[pl.ds(f * VPAD, VOCAB)] = x_ref[0, f, :]


def _untile(tables_ev):
    return pl.pallas_call(
        _untile_body,
        grid=(EMB,),
        in_specs=[pl.BlockSpec((1, NUM_FIELDS, VOCAB), lambda e: (e, 0, 0))],
        out_specs=pl.BlockSpec((ESTRIDE,), lambda e: (e,)),
        out_shape=jax.ShapeDtypeStruct((NWORDS,), jnp.float32),
    )(tables_ev)


def _dense_body(x_ref, w_ref, b_ref, o_ref):
    acc = lax.dot_general(
        x_ref[:, :],
        w_ref[:, :],
        dimension_numbers=(((1,), (1,)), ((), ())),
        preferred_element_type=jnp.float32,
    )
    o_ref[:, :] = acc + b_ref[:, :]


def _dense(num_feat, W, b):
    return pl.pallas_call(
        _dense_body,
        out_shape=jax.ShapeDtypeStruct((B, NUM_DENSE), jnp.float32),
    )(num_feat, W, b.reshape(1, NUM_DENSE))


def kernel(cate_feat, num_feat, tables, W, b):
    ftab = _untile(jnp.transpose(tables, (2, 0, 1)))
    idx_flat = cate_feat.astype(jnp.int32).T.reshape(-1)
    emb = _sc_gather(ftab, idx_flat)
    num_out = _dense(num_feat, W, b)
    return jnp.concatenate(
        [emb.reshape(B, NUM_FIELDS * EMB), num_out], axis=1)


# speedup vs baseline: 1.0802x; 1.0077x over previous
import functools

import jax
import jax.numpy as jnp
from jax import lax
from jax.experimental import pallas as pl
from jax.experimental.pallas import tpu as pltpu
from jax.experimental.pallas import tpu_sc as plsc

NUM_FIELDS = 26
VOCAB = 100000
EMB = 5
B = 4096
NUM_DENSE = 13

NUM_CORES = 2
NUM_WORKERS = 32
BPW = B // NUM_WORKERS
JPW = BPW * NUM_FIELDS
VPAD = 100352
FSTRIDE = VPAD
ESTRIDE = NUM_FIELDS * VPAD
NWORDS = EMB * ESTRIDE
CHUNK = 128
OPW = JPW * EMB
NGATHER = OPW // CHUNK
NIDXV = JPW // 16
NOUTV = OPW // 16


def _sc_gather_body(ftab_hbm, idx_hbm, out_hbm, idxv, eidx, vals, sem):
    wid = lax.axis_index("s") * NUM_CORES + lax.axis_index("c")
    base = wid * JPW

    pltpu.sync_copy(idx_hbm.at[pl.ds(base, JPW)], idxv)

    lanes = lax.iota(jnp.int32, 16)

    def add_offsets(i, _):
        off16 = i * 16
        j = lanes + off16
        f = lax.rem(j, NUM_FIELDS)
        idxv[pl.ds(off16, 16)] = idxv[pl.ds(off16, 16)] + f * FSTRIDE
        return 0

    lax.fori_loop(0, NIDXV, add_offsets, 0)

    def expand(i, _):
        p0 = i * 16
        p = lanes + p0
        j = lax.div(p, EMB)
        e = p - j * EMB
        eidx[pl.ds(p0, 16)] = plsc.load_gather(idxv, [j]) + e * ESTRIDE
        return 0

    NWAVE = 5
    copies = []
    half = NOUTV // NWAVE
    for g in range(NWAVE):
        lax.fori_loop(g * half, (g + 1) * half, expand, 0)
        for k in range(g * NGATHER // NWAVE, (g + 1) * NGATHER // NWAVE):
            copies.append(
                pltpu.async_copy(
                    ftab_hbm.at[eidx.at[pl.ds(k * CHUNK, CHUNK)]],
                    vals.at[pl.ds(k * CHUNK, CHUNK)],
                    sem,
                )
            )
    for c in copies:
        c.wait()

    pltpu.sync_copy(vals, out_hbm.at[pl.ds(wid * OPW, OPW)])


def _sc_gather(ftab, idx_flat):
    mesh = plsc.VectorSubcoreMesh(core_axis_name="c", subcore_axis_name="s")
    kern = functools.partial(
        pl.kernel,
        mesh=mesh,
        out_type=jax.ShapeDtypeStruct((B * NUM_FIELDS * EMB,), jnp.float32),
        scratch_types=[
            pltpu.VMEM((JPW,), jnp.int32),
            pltpu.VMEM((OPW,), jnp.int32),
            pltpu.VMEM((OPW,), jnp.float32),
            pltpu.SemaphoreType.DMA,
        ],
        compiler_params=pltpu.CompilerParams(
            use_tc_tiling_on_sc=False, needs_layout_passes=False),
    )(_sc_gather_body)
    return kern(ftab, idx_flat)


def _untile_body(x_ref, o_ref):
    for f in range(NUM_FIELDS):
        o_ref[pl.ds(f * VPAD, VOCAB)] = x_ref[0, f, :]


def _untile(tables_ev):
    return pl.pallas_call(
        _untile_body,
        grid=(EMB,),
        in_specs=[pl.BlockSpec((1, NUM_FIELDS, VOCAB), lambda e: (e, 0, 0))],
        out_specs=pl.BlockSpec((ESTRIDE,), lambda e: (e,)),
        out_shape=jax.ShapeDtypeStruct((NWORDS,), jnp.float32),
    )(tables_ev)


def _dense_body(x_ref, w_ref, b_ref, o_ref):
    acc = lax.dot_general(
        x_ref[:, :],
        w_ref[:, :],
        dimension_numbers=(((1,), (1,)), ((), ())),
        preferred_element_type=jnp.float32,
    )
    o_ref[:, :] = acc + b_ref[:, :]


def _dense(num_feat, W, b):
    return pl.pallas_call(
        _dense_body,
        out_shape=jax.ShapeDtypeStruct((B, NUM_DENSE), jnp.float32),
    )(num_feat, W, b.reshape(1, NUM_DENSE))


def kernel(cate_feat, num_feat, tables, W, b):
    ftab = _untile(jnp.transpose(tables, (2, 0, 1)))
    idx_flat = cate_feat.astype(jnp.int32).T.reshape(-1)
    emb = _sc_gather(ftab, idx_flat)
    num_out = _dense(num_feat, W, b)
    return jnp.concatenate(
        [emb.reshape(B, NUM_FIELDS * EMB), num_out], axis=1)
